# Initial kernel scaffold; baseline (speedup 1.0000x reference)
import jax
import jax.numpy as jnp
from jax.experimental import pallas as pl


def kernel(node_from, node_to, edge_lengths, edge_vectors, node_graph_index, num_nodes, num_graphs, W_dot, b_dot, W_cross, b_cross, W_vec, b_vec, W_out, b_out):
    raise NotImplementedError("write your pallas kernel here")


# speedup vs baseline: 72.6836x; 72.6836x over previous
import functools

import jax
import jax.numpy as jnp
from jax import lax
from jax.experimental import pallas as pl
from jax.experimental.pallas import tpu as pltpu
from jax.experimental.pallas import tpu_sc as plsc

NC, NS, L = 2, 16, 16
NW = NC * NS
CH = 128
S = 10

_mesh = plsc.VectorSubcoreMesh(
    core_axis_name="c", subcore_axis_name="s", num_cores=NC, num_subcores=NS)
_params = pltpu.CompilerParams(use_tc_tiling_on_sc=False)


def _tanh(x):
    e = jnp.exp(x + x)
    return 1.0 - 2.0 / (e + 1.0)


def _zero_acc(zer_h, acc, r0, rpt):
    crows = rpt // 8
    for c in range(8):
        pltpu.sync_copy(zer_h.at[pl.ds(c * crows, crows)],
                        acc.at[pl.ds(r0 + c * crows, crows)])


def _read_acc(acc, out_h, r0, rpt, obase):
    crows = rpt // 8
    for c in range(8):
        pltpu.sync_copy(acc.at[pl.ds(r0 + c * crows, crows)],
                        out_h.at[pl.ds(obase + r0 + c * crows, crows)])


def _pass_a(to1d, lens, v, wb, zer, n_rows):
    E = lens.shape[0]
    nch = E // CH

    @functools.partial(
        pl.kernel,
        out_type=jax.ShapeDtypeStruct((NC * n_rows, L), jnp.float32),
        mesh=_mesh,
        compiler_params=_params,
        scratch_types=[
            pltpu.VMEM((CH,), jnp.int32),
            pltpu.VMEM((CH,), jnp.float32),
            pltpu.VMEM((CH,), jnp.float32),
            pltpu.VMEM((2 * L,), jnp.float32),
            pltpu.VMEM((CH, L), jnp.float32),
            pltpu.VMEM_SHARED((n_rows, L), jnp.float32),
        ],
    )
    def k(to_h, len_h, v_h, wb_h, zer_h, out_h,
          to_b, len_b, v_b, wb_b, msg_b, acc):
        cid = lax.axis_index("c")
        sid = lax.axis_index("s")
        wid = sid * NC + cid
        rpt = n_rows // NS
        r0 = sid * rpt
        _zero_acc(zer_h, acc, r0, rpt)
        pltpu.sync_copy(wb_h, wb_b)
        plsc.subcore_barrier()

        n_i = (nch - wid + NW - 1) // NW

        def chunk(i, carry):
            e0 = (wid + i * NW) * CH
            pltpu.sync_copy(to_h.at[pl.ds(e0, CH)], to_b)
            pltpu.sync_copy(len_h.at[pl.ds(e0, CH)], len_b)
            pltpu.sync_copy(v_h.at[pl.ds(e0, CH)], v_b)
            wv = wb_b[pl.ds(0, L)]
            bv = wb_b[pl.ds(L, L)]

            def grp(gg, c2):
                off = gg * L
                lvec = len_b[pl.ds(off, L)]
                vv = v_b[pl.ds(off, L)]
                for e in range(L):
                    t = _tanh(lvec[e] * wv + bv)
                    msg_b[off + e, pl.ds(0, L)] = t * vv[e]
                return c2

            lax.fori_loop(0, CH // L, grp, 0, unroll=False)
            pltpu.sync_copy(msg_b, acc.at[to_b], add=True)
            return carry

        lax.fori_loop(0, n_i, chunk, 0, unroll=False)
        plsc.subcore_barrier()
        _read_acc(acc, out_h, r0, rpt, cid * n_rows)

    return k(to1d, lens, v, wb, zer)


def _pass_b(fr1d, to1d, lens, tsv, wb2, zer, n_rows):
    E = lens.shape[0]
    nch = E // CH

    @functools.partial(
        pl.kernel,
        out_type=jax.ShapeDtypeStruct((NC * n_rows, L), jnp.float32),
        mesh=_mesh,
        compiler_params=_params,
        scratch_types=[
            pltpu.VMEM((CH,), jnp.int32),
            pltpu.VMEM((CH,), jnp.int32),
            pltpu.VMEM((CH,), jnp.float32),
            pltpu.VMEM((4 * L,), jnp.float32),
            pltpu.VMEM((CH, 2 * L), jnp.float32),
            pltpu.VMEM((CH, 2 * L), jnp.float32),
            pltpu.VMEM((CH, L), jnp.float32),
            pltpu.VMEM_SHARED((n_rows, L), jnp.float32),
            pltpu.SemaphoreType.DMA,
        ],
    )
    def k(fr_h, to_h, len_h, tsv_h, wb_h, zer_h, out_h,
          fr_b, to_b, len_b, wb_b, f_b, t_b, msg_b, acc, sem):
        cid = lax.axis_index("c")
        sid = lax.axis_index("s")
        wid = sid * NC + cid
        rpt = n_rows // NS
        r0 = sid * rpt
        _zero_acc(zer_h, acc, r0, rpt)
        pltpu.sync_copy(wb_h, wb_b)
        plsc.subcore_barrier()

        n_i = (nch - wid + NW - 1) // NW

        def chunk(i, carry):
            e0 = (wid + i * NW) * CH
            pltpu.sync_copy(fr_h.at[pl.ds(e0, CH)], fr_b)
            pltpu.sync_copy(to_h.at[pl.ds(e0, CH)], to_b)
            pltpu.sync_copy(len_h.at[pl.ds(e0, CH)], len_b)
            pltpu.async_copy(tsv_h.at[fr_b], f_b, sem).wait()
            pltpu.async_copy(tsv_h.at[to_b], t_b, sem).wait()
            wd = wb_b[pl.ds(0, L)]
            bd = wb_b[pl.ds(L, L)]
            wc = wb_b[pl.ds(2 * L, L)]
            bc = wb_b[pl.ds(3 * L, L)]

            def grp(gg, c2):
                off = gg * L
                lvec = len_b[pl.ds(off, L)]
                for e in range(L):
                    row = off + e
                    le = lvec[e]
                    svxf = f_b[row, pl.ds(0, L)]
                    svyf = f_b[row, pl.ds(L, L)]
                    svxt = t_b[row, pl.ds(0, L)]
                    svyt = t_b[row, pl.ds(L, L)]
                    md0 = _tanh(le * wd + bd)
                    mc0 = _tanh(le * wc + bc)
                    dot2 = svxf * svxt + svyf * svyt
                    cr2 = svxf * svyt - svyf * svxt
                    msg_b[row, pl.ds(0, L)] = md0 * dot2 + mc0 * cr2
                return c2

            lax.fori_loop(0, CH // L, grp, 0, unroll=False)
            pltpu.sync_copy(msg_b, acc.at[to_b], add=True)
            return carry

        lax.fori_loop(0, n_i, chunk, 0, unroll=False)
        plsc.subcore_barrier()
        _read_acc(acc, out_h, r0, rpt, cid * n_rows)

    return k(fr1d, to1d, lens, tsv, wb2, zer)


def _pass_c(fr1d, to1d, lens, tfrom, tsv, wm, n_rows, G):
    E = lens.shape[0]
    nch = E // CH

    @functools.partial(
        pl.kernel,
        out_type=jax.ShapeDtypeStruct((NW * G, L), jnp.float32),
        mesh=_mesh,
        compiler_params=_params,
        scratch_types=[
            pltpu.VMEM((CH,), jnp.int32),
            pltpu.VMEM((CH,), jnp.int32),
            pltpu.VMEM((CH,), jnp.float32),
            pltpu.VMEM((24 * L,), jnp.float32),
            pltpu.VMEM((CH, 3 * L), jnp.float32),
            pltpu.VMEM((CH, 2 * L), jnp.float32),
            pltpu.VMEM((16, L), jnp.float32),
            pltpu.SemaphoreType.DMA,
        ],
    )
    def k(fr_h, to_h, len_h, tfrom_h, tsv_h, wm_h, out_h,
          fr_b, to_b, len_b, wm_b, f_b, t_b, gacc, sem):
        cid = lax.axis_index("c")
        sid = lax.axis_index("s")
        wid = sid * NC + cid
        pltpu.sync_copy(wm_h, wm_b)
        zero = jnp.zeros((L,), jnp.float32)
        for g in range(G):
            gacc[g, pl.ds(0, L)] = zero

        wdk = [wm_b[pl.ds(kk * L, L)] for kk in range(S + 1)]
        bd = wm_b[pl.ds(11 * L, L)]
        wck = [wm_b[pl.ds((12 + kk) * L, L)] for kk in range(S + 1)]
        bc = wm_b[pl.ds(23 * L, L)]

        n_i = (nch - wid + NW - 1) // NW

        def chunk(i, carry):
            e0 = (wid + i * NW) * CH
            pltpu.sync_copy(fr_h.at[pl.ds(e0, CH)], fr_b)
            pltpu.sync_copy(to_h.at[pl.ds(e0, CH)], to_b)
            pltpu.sync_copy(len_h.at[pl.ds(e0, CH)], len_b)
            pltpu.async_copy(tfrom_h.at[fr_b], f_b, sem).wait()
            pltpu.async_copy(tsv_h.at[to_b], t_b, sem).wait()

            def grp(gg, c2):
                off = gg * L
                lvec = len_b[pl.ds(off, L)]
                for e in range(L):
                    row = off + e
                    le = lvec[e]
                    st2 = f_b[row, pl.ds(0, L)]
                    svxf = f_b[row, pl.ds(L, L)]
                    svyf = f_b[row, pl.ds(2 * L, L)]
                    svxt = t_b[row, pl.ds(0, L)]
                    svyt = t_b[row, pl.ds(L, L)]
                    accd = le * wdk[10] + bd
                    accc = le * wck[10] + bc
                    for kk in range(S):
                        sk = st2[kk]
                        accd = accd + sk * wdk[kk]
                        accc = accc + sk * wck[kk]
                    md3 = _tanh(accd)
                    mc3 = _tanh(accc)
                    dot2 = svxf * svxt + svyf * svyt
                    cr2 = svxf * svyt - svyf * svxt
                    msg = (md3 * dot2 + mc3 * cr2) * 4.0
                    gi = svxt[15].astype(jnp.int32)
                    gacc[gi, pl.ds(0, L)] = gacc[gi, pl.ds(0, L)] + msg
                return c2

            lax.fori_loop(0, CH // L, grp, 0, unroll=False)
            return carry

        lax.fori_loop(0, n_i, chunk, 0, unroll=False)
        pltpu.sync_copy(gacc, out_h.at[pl.ds(wid * G, G)])

    return k(fr1d, to1d, lens, tfrom, tsv, wm)


def _pad16(v):
    return jnp.zeros((L,), jnp.float32).at[:S].set(v)


def kernel(node_from, node_to, edge_lengths, edge_vectors, node_graph_index,
           num_nodes, num_graphs, W_dot, b_dot, W_cross, b_cross,
           W_vec, b_vec, W_out, b_out):
    E = node_from.shape[0]
    N = node_graph_index.shape[0]
    G = 16
    n_rows = ((N + NS * 8 - 1) // (NS * 8)) * (NS * 8)

    fr1d = node_from.astype(jnp.int32)
    to1d = node_to.astype(jnp.int32)
    lens = edge_lengths[:, 0]
    vx = edge_vectors[:, 0]
    vy = edge_vectors[:, 1]
    zer = jnp.zeros((n_rows // NS, L), jnp.float32)

    wba = jnp.concatenate([_pad16(W_vec[S]), _pad16(b_vec)])
    px = _pass_a(to1d, lens, vx, wba, zer, n_rows)
    py = _pass_a(to1d, lens, vy, wba, zer, n_rows)
    svx = px[:n_rows] + px[n_rows:]
    svy = py[:n_rows] + py[n_rows:]

    gcol = jnp.zeros((n_rows,), jnp.float32).at[:N].set(
        node_graph_index.astype(jnp.float32))
    tsv = jnp.concatenate([svx.at[:, 15].set(gcol), svy], axis=1)

    wb2 = jnp.concatenate([_pad16(W_dot[S]), _pad16(b_dot),
                           _pad16(W_cross[S]), _pad16(b_cross)])
    pb = _pass_b(fr1d, to1d, lens, tsv, wb2, zer, n_rows)
    state2 = pb[:n_rows] + pb[n_rows:]

    tfrom = jnp.concatenate([state2, svx, svy], axis=1)

    wm = jnp.concatenate(
        [_pad16(W_dot[k]) for k in range(S + 1)] + [_pad16(b_dot)]
        + [_pad16(W_cross[k]) for k in range(S + 1)] + [_pad16(b_cross)])
    pc = _pass_c(fr1d, to1d, lens, tfrom, tsv, wm, n_rows, G)
    graph_msg3 = jnp.sum(pc.reshape(NW, G, L), axis=0)

    graph_state2 = jax.ops.segment_sum(
        state2[:N], node_graph_index, num_segments=G)
    graph3 = (graph_state2 + graph_msg3)[:, :S]
    return graph3 @ W_out + b_out


# speedup vs baseline: 113.7694x; 1.5653x over previous
import functools

import jax
import jax.numpy as jnp
from jax import lax
from jax.experimental import pallas as pl
from jax.experimental.pallas import tpu as pltpu
from jax.experimental.pallas import tpu_sc as plsc

NC, NS, L = 2, 16, 16
NW = NC * NS
CH = 128
S = 10

_mesh = plsc.VectorSubcoreMesh(
    core_axis_name="c", subcore_axis_name="s", num_cores=NC, num_subcores=NS)
_params = pltpu.CompilerParams(use_tc_tiling_on_sc=False)


def _tanh(x):
    e = jnp.exp(x + x)
    return 1.0 - 2.0 / (e + 1.0)


def _zero_acc(zer_h, acc, r0, rpt):
    crows = rpt // 8
    for c in range(8):
        pltpu.sync_copy(zer_h.at[pl.ds(c * crows, crows)],
                        acc.at[pl.ds(r0 + c * crows, crows)])


def _read_acc(acc, out_h, r0, rpt, obase):
    crows = rpt // 8
    for c in range(8):
        pltpu.sync_copy(acc.at[pl.ds(r0 + c * crows, crows)],
                        out_h.at[pl.ds(obase + r0 + c * crows, crows)])


def _pass_a(to1d, lens, v, wb, zer, n_rows, nloc):
    @functools.partial(
        pl.kernel,
        out_type=jax.ShapeDtypeStruct((NC * n_rows, L), jnp.float32),
        mesh=_mesh,
        compiler_params=_params,
        scratch_types=[
            pltpu.VMEM((CH,), jnp.int32),
            pltpu.VMEM((CH,), jnp.int32),
            pltpu.VMEM((CH,), jnp.float32),
            pltpu.VMEM((CH,), jnp.float32),
            pltpu.VMEM((CH,), jnp.float32),
            pltpu.VMEM((CH,), jnp.float32),
            pltpu.VMEM((2 * L,), jnp.float32),
            pltpu.VMEM((CH, L), jnp.float32),
            pltpu.VMEM_SHARED((n_rows, L), jnp.float32),
            pltpu.SemaphoreType.DMA,
        ],
    )
    def k(to_h, len_h, v_h, wb_h, zer_h, out_h,
          to_b0, to_b1, len_b0, len_b1, v_b0, v_b1, wb_b, msg_b, acc, ssem):
        cid = lax.axis_index("c")
        sid = lax.axis_index("s")
        wid = sid * NC + cid
        rpt = n_rows // NS
        r0 = sid * rpt
        _zero_acc(zer_h, acc, r0, rpt)
        pltpu.sync_copy(wb_h, wb_b)
        plsc.subcore_barrier()
        wv = wb_b[pl.ds(0, L)]
        bv = wb_b[pl.ds(L, L)]
        bufs = ((to_b0, len_b0, v_b0), (to_b1, len_b1, v_b1))

        def stage(slot, ci):
            e0 = (wid + jnp.minimum(ci, nloc - 1) * NW) * CH
            to_b, len_b, v_b = bufs[slot]
            pltpu.async_copy(to_h.at[pl.ds(e0, CH)], to_b, ssem)
            pltpu.async_copy(len_h.at[pl.ds(e0, CH)], len_b, ssem)
            pltpu.async_copy(v_h.at[pl.ds(e0, CH)], v_b, ssem)

        def wait_stage(slot):
            to_b, len_b, v_b = bufs[slot]
            pltpu.make_async_copy(to_h.at[pl.ds(0, CH)], to_b, ssem).wait()
            pltpu.make_async_copy(len_h.at[pl.ds(0, CH)], len_b, ssem).wait()
            pltpu.make_async_copy(v_h.at[pl.ds(0, CH)], v_b, ssem).wait()

        def compute(slot):
            to_b, len_b, v_b = bufs[slot]

            def grp(gg, c2):
                off = gg * L
                lvec = len_b[pl.ds(off, L)]
                vv = v_b[pl.ds(off, L)]
                for e in range(L):
                    t = _tanh(lvec[e] * wv + bv)
                    msg_b[off + e, pl.ds(0, L)] = t * vv[e]
                return c2

            lax.fori_loop(0, CH // L, grp, 0, unroll=False)
            pltpu.sync_copy(msg_b, acc.at[to_b], add=True)

        stage(0, 0)
        stage(1, 1)

        def body(g, carry):
            c0 = g * 2
            for slot in (0, 1):
                wait_stage(slot)
                compute(slot)
                stage(slot, c0 + slot + 2)
            return carry

        lax.fori_loop(0, nloc // 2, body, 0, unroll=False)
        wait_stage(0)
        wait_stage(1)
        plsc.subcore_barrier()
        _read_acc(acc, out_h, r0, rpt, cid * n_rows)

    return k(to1d, lens, v, wb, zer)


def _pass_b(fr1d, to1d, lens, tsv, wb2, zer, n_rows, nloc):
    @functools.partial(
        pl.kernel,
        out_type=jax.ShapeDtypeStruct((NC * n_rows, L), jnp.float32),
        mesh=_mesh,
        compiler_params=_params,
        scratch_types=[
            pltpu.VMEM((CH,), jnp.int32),
            pltpu.VMEM((CH,), jnp.int32),
            pltpu.VMEM((CH,), jnp.int32),
            pltpu.VMEM((CH,), jnp.int32),
            pltpu.VMEM((CH,), jnp.float32),
            pltpu.VMEM((CH,), jnp.float32),
            pltpu.VMEM((4 * L,), jnp.float32),
            pltpu.VMEM((CH, 2 * L), jnp.float32),
            pltpu.VMEM((CH, 2 * L), jnp.float32),
            pltpu.VMEM((CH, L), jnp.float32),
            pltpu.VMEM_SHARED((n_rows, L), jnp.float32),
            pltpu.SemaphoreType.DMA,
            pltpu.SemaphoreType.DMA,
        ],
    )
    def k(fr_h, to_h, len_h, tsv_h, wb_h, zer_h, out_h,
          fr_b0, fr_b1, to_b0, to_b1, len_b0, len_b1,
          wb_b, f_b, t_b, msg_b, acc, ssem, gsem):
        cid = lax.axis_index("c")
        sid = lax.axis_index("s")
        wid = sid * NC + cid
        rpt = n_rows // NS
        r0 = sid * rpt
        _zero_acc(zer_h, acc, r0, rpt)
        pltpu.sync_copy(wb_h, wb_b)
        plsc.subcore_barrier()
        wd = wb_b[pl.ds(0, L)]
        bd = wb_b[pl.ds(L, L)]
        wc = wb_b[pl.ds(2 * L, L)]
        bc = wb_b[pl.ds(3 * L, L)]
        bufs = ((fr_b0, to_b0, len_b0), (fr_b1, to_b1, len_b1))

        def stage(slot, ci):
            e0 = (wid + jnp.minimum(ci, nloc - 1) * NW) * CH
            fr_b, to_b, len_b = bufs[slot]
            pltpu.async_copy(fr_h.at[pl.ds(e0, CH)], fr_b, ssem)
            pltpu.async_copy(to_h.at[pl.ds(e0, CH)], to_b, ssem)
            pltpu.async_copy(len_h.at[pl.ds(e0, CH)], len_b, ssem)

        def wait_stage(slot):
            fr_b, to_b, len_b = bufs[slot]
            pltpu.make_async_copy(fr_h.at[pl.ds(0, CH)], fr_b, ssem).wait()
            pltpu.make_async_copy(to_h.at[pl.ds(0, CH)], to_b, ssem).wait()
            pltpu.make_async_copy(len_h.at[pl.ds(0, CH)], len_b, ssem).wait()

        stage(0, 0)
        stage(1, 1)

        def body(g, carry):
            c0 = g * 2
            for slot in (0, 1):
                fr_b, to_b, len_b = bufs[slot]
                wait_stage(slot)
                ga = pltpu.async_copy(tsv_h.at[fr_b], f_b, gsem)
                gb = pltpu.async_copy(tsv_h.at[to_b], t_b, gsem)
                ga.wait()
                gb.wait()

                def grp(gg, c2):
                    off = gg * L
                    lvec = len_b[pl.ds(off, L)]
                    for e in range(L):
                        row = off + e
                        le = lvec[e]
                        svxf = f_b[row, pl.ds(0, L)]
                        svyf = f_b[row, pl.ds(L, L)]
                        svxt = t_b[row, pl.ds(0, L)]
                        svyt = t_b[row, pl.ds(L, L)]
                        md0 = _tanh(le * wd + bd)
                        mc0 = _tanh(le * wc + bc)
                        dot2 = svxf * svxt + svyf * svyt
                        cr2 = svxf * svyt - svyf * svxt
                        msg_b[row, pl.ds(0, L)] = md0 * dot2 + mc0 * cr2
                    return c2

                lax.fori_loop(0, CH // L, grp, 0, unroll=False)
                pltpu.sync_copy(msg_b, acc.at[to_b], add=True)
                stage(slot, c0 + slot + 2)
            return carry

        lax.fori_loop(0, nloc // 2, body, 0, unroll=False)
        wait_stage(0)
        wait_stage(1)
        plsc.subcore_barrier()
        _read_acc(acc, out_h, r0, rpt, cid * n_rows)

    return k(fr1d, to1d, lens, tsv, wb2, zer)


def _pass_c(fr1d, to1d, lens, tfrom, tsv, wm, n_rows, G, nloc):
    @functools.partial(
        pl.kernel,
        out_type=jax.ShapeDtypeStruct((NW * G, L), jnp.float32),
        mesh=_mesh,
        compiler_params=_params,
        scratch_types=[
            pltpu.VMEM((CH,), jnp.int32),
            pltpu.VMEM((CH,), jnp.int32),
            pltpu.VMEM((CH,), jnp.int32),
            pltpu.VMEM((CH,), jnp.int32),
            pltpu.VMEM((CH,), jnp.float32),
            pltpu.VMEM((CH,), jnp.float32),
            pltpu.VMEM((24 * L,), jnp.float32),
            pltpu.VMEM((CH, 3 * L), jnp.float32),
            pltpu.VMEM((CH, 3 * L), jnp.float32),
            pltpu.VMEM((CH, 2 * L), jnp.float32),
            pltpu.VMEM((CH, 2 * L), jnp.float32),
            pltpu.VMEM((16, L), jnp.float32),
            pltpu.SemaphoreType.DMA,
            pltpu.SemaphoreType.DMA,
        ],
    )
    def k(fr_h, to_h, len_h, tfrom_h, tsv_h, wm_h, out_h,
          fr_b0, fr_b1, to_b0, to_b1, len_b0, len_b1,
          wm_b, f_b0, f_b1, t_b0, t_b1, gacc, ssem, gsem):
        cid = lax.axis_index("c")
        sid = lax.axis_index("s")
        wid = sid * NC + cid
        pltpu.sync_copy(wm_h, wm_b)
        zero = jnp.zeros((L,), jnp.float32)
        for g in range(G):
            gacc[g, pl.ds(0, L)] = zero

        wdk = [wm_b[pl.ds(kk * L, L)] for kk in range(S + 1)]
        bd = wm_b[pl.ds(11 * L, L)]
        wck = [wm_b[pl.ds((12 + kk) * L, L)] for kk in range(S + 1)]
        bc = wm_b[pl.ds(23 * L, L)]
        sbufs = ((fr_b0, to_b0, len_b0), (fr_b1, to_b1, len_b1))
        gbufs = ((f_b0, t_b0), (f_b1, t_b1))

        def stage(slot, ci):
            e0 = (wid + jnp.minimum(ci, nloc - 1) * NW) * CH
            fr_b, to_b, len_b = sbufs[slot]
            pltpu.async_copy(fr_h.at[pl.ds(e0, CH)], fr_b, ssem)
            pltpu.async_copy(to_h.at[pl.ds(e0, CH)], to_b, ssem)
            pltpu.async_copy(len_h.at[pl.ds(e0, CH)], len_b, ssem)

        def wait_stage(slot):
            fr_b, to_b, len_b = sbufs[slot]
            pltpu.make_async_copy(fr_h.at[pl.ds(0, CH)], fr_b, ssem).wait()
            pltpu.make_async_copy(to_h.at[pl.ds(0, CH)], to_b, ssem).wait()
            pltpu.make_async_copy(len_h.at[pl.ds(0, CH)], len_b, ssem).wait()

        def gath(slot):
            fr_b, to_b, _ = sbufs[slot]
            f_b, t_b = gbufs[slot]
            pltpu.async_copy(tfrom_h.at[fr_b], f_b, gsem)
            pltpu.async_copy(tsv_h.at[to_b], t_b, gsem)

        def wait_gath(slot):
            fr_b, to_b, _ = sbufs[slot]
            f_b, t_b = gbufs[slot]
            pltpu.make_async_copy(tfrom_h.at[fr_b], f_b, gsem).wait()
            pltpu.make_async_copy(tsv_h.at[to_b], t_b, gsem).wait()

        def compute(slot):
            _, _, len_b = sbufs[slot]
            f_b, t_b = gbufs[slot]

            def grp(gg, c2):
                off = gg * L
                lvec = len_b[pl.ds(off, L)]
                for e in range(L):
                    row = off + e
                    le = lvec[e]
                    st2 = f_b[row, pl.ds(0, L)]
                    svxf = f_b[row, pl.ds(L, L)]
                    svyf = f_b[row, pl.ds(2 * L, L)]
                    svxt = t_b[row, pl.ds(0, L)]
                    svyt = t_b[row, pl.ds(L, L)]
                    accd = le * wdk[10] + bd
                    accc = le * wck[10] + bc
                    for kk in range(S):
                        sk = st2[kk]
                        accd = accd + sk * wdk[kk]
                        accc = accc + sk * wck[kk]
                    md3 = _tanh(accd)
                    mc3 = _tanh(accc)
                    dot2 = svxf * svxt + svyf * svyt
                    cr2 = svxf * svyt - svyf * svxt
                    msg = (md3 * dot2 + mc3 * cr2) * 4.0
                    gi = svxt[15].astype(jnp.int32)
                    gacc[gi, pl.ds(0, L)] = gacc[gi, pl.ds(0, L)] + msg
                return c2

            lax.fori_loop(0, CH // L, grp, 0, unroll=False)

        stage(0, 0)
        wait_stage(0)
        gath(0)
        stage(1, 1)

        def body(g, carry):
            c0 = g * 2
            wait_stage(1)
            gath(1)
            wait_gath(0)
            compute(0)
            stage(0, c0 + 2)
            wait_gath(1)
            compute(1)
            stage(1, c0 + 3)
            wait_stage(0)
            gath(0)
            return carry

        lax.fori_loop(0, nloc // 2, body, 0, unroll=False)
        wait_gath(0)
        wait_stage(1)
        pltpu.sync_copy(gacc, out_h.at[pl.ds(wid * G, G)])

    return k(fr1d, to1d, lens, tfrom, tsv, wm)


def _pad16(v):
    return jnp.zeros((L,), jnp.float32).at[:S].set(v)


def kernel(node_from, node_to, edge_lengths, edge_vectors, node_graph_index,
           num_nodes, num_graphs, W_dot, b_dot, W_cross, b_cross,
           W_vec, b_vec, W_out, b_out):
    E = node_from.shape[0]
    N = node_graph_index.shape[0]
    G = 16
    n_rows = ((N + NS * 8 - 1) // (NS * 8)) * (NS * 8)

    nchunks = -(-E // CH)
    nloc = -(-nchunks // NW)
    nloc += nloc % 2
    Ep = NW * nloc * CH
    pad = Ep - E

    def _padi(a, v):
        return jnp.concatenate([a, jnp.full((pad,), v, a.dtype)])

    fr1d = _padi(node_from.astype(jnp.int32), N)
    to1d = _padi(node_to.astype(jnp.int32), N)
    lens = _padi(edge_lengths[:, 0], 0)
    vx = _padi(edge_vectors[:, 0], 0)
    vy = _padi(edge_vectors[:, 1], 0)
    zer = jnp.zeros((n_rows // NS, L), jnp.float32)

    wba = jnp.concatenate([_pad16(W_vec[S]), _pad16(b_vec)])
    px = _pass_a(to1d, lens, vx, wba, zer, n_rows, nloc)
    py = _pass_a(to1d, lens, vy, wba, zer, n_rows, nloc)
    svx = px[:n_rows] + px[n_rows:]
    svy = py[:n_rows] + py[n_rows:]

    gcol = jnp.zeros((n_rows,), jnp.float32).at[:N].set(
        node_graph_index.astype(jnp.float32))
    tsv = jnp.concatenate([svx.at[:, 15].set(gcol), svy], axis=1)

    wb2 = jnp.concatenate([_pad16(W_dot[S]), _pad16(b_dot),
                           _pad16(W_cross[S]), _pad16(b_cross)])
    pb = _pass_b(fr1d, to1d, lens, tsv, wb2, zer, n_rows, nloc)
    state2 = pb[:n_rows] + pb[n_rows:]

    tfrom = jnp.concatenate([state2, svx, svy], axis=1)

    wm = jnp.concatenate(
        [_pad16(W_dot[k]) for k in range(S + 1)] + [_pad16(b_dot)]
        + [_pad16(W_cross[k]) for k in range(S + 1)] + [_pad16(b_cross)])
    pc = _pass_c(fr1d, to1d, lens, tfrom, tsv, wm, n_rows, G, nloc)
    graph_msg3 = jnp.sum(pc.reshape(NW, G, L), axis=0)

    graph_state2 = jax.ops.segment_sum(
        state2[:N], node_graph_index, num_segments=G)
    graph3 = (graph_state2 + graph_msg3)[:, :S]
    return graph3 @ W_out + b_out
